# baseline (speedup 1.0000x reference)
import jax
import jax.numpy as jnp
from jax import lax
from jax.experimental import pallas as pl
from jax.experimental.pallas import tpu as pltpu
from jax.experimental.pallas import tpu_sc as plsc

NUM_SPARSE = 26
HIST_LEN = 50
VOCAB = 1000000
BATCH = 16384
NUM_FIELDS = NUM_SPARSE + HIST_LEN

L = 16
NW = 32
B_PER_W = BATCH // NW


def _logit_kernel(inputs_t_hbm, tables_t_hbm, out_hbm,
                  idx_t, vals, outbuf, sem):
    wid = lax.axis_index("s") * 2 + lax.axis_index("c")
    base = wid * B_PER_W

    pltpu.sync_copy(inputs_t_hbm.at[:, pl.ds(base, B_PER_W)], idx_t)

    copies = []
    for c in range(NUM_FIELDS):
        t = min(c, NUM_SPARSE)
        copies.append(pltpu.async_copy(
            tables_t_hbm.at[t, 0, :].at[idx_t.at[c]],
            vals.at[c],
            sem))
    for cp in copies:
        cp.wait()

    def rbody(v, carry):
        o = v * L
        acc = jnp.zeros((L,), jnp.float32)
        for c in range(NUM_SPARSE):
            acc = acc + vals[c, pl.ds(o, L)]
        for c in range(NUM_SPARSE, NUM_FIELDS):
            val = vals[c, pl.ds(o, L)]
            raw = idx_t[c, pl.ds(o, L)]
            acc = acc + jnp.where(raw != 0, val, 0.0)
        outbuf[pl.ds(o, L)] = acc
        return carry

    lax.fori_loop(0, B_PER_W // L, rbody, 0)

    pltpu.sync_copy(outbuf, out_hbm.at[pl.ds(base, B_PER_W)])


@jax.jit
def _run(inputs_t, tables_t):
    mesh = plsc.VectorSubcoreMesh(core_axis_name="c", subcore_axis_name="s")
    return pl.kernel(
        _logit_kernel,
        mesh=mesh,
        compiler_params=pltpu.CompilerParams(
            needs_layout_passes=False, use_tc_tiling_on_sc=False),
        out_type=jax.ShapeDtypeStruct((BATCH,), jnp.float32),
        scratch_types=[
            pltpu.VMEM((NUM_FIELDS, B_PER_W), jnp.int32),
            pltpu.VMEM((NUM_FIELDS, B_PER_W), jnp.float32),
            pltpu.VMEM((B_PER_W,), jnp.float32),
            pltpu.SemaphoreType.DMA,
        ],
    )(inputs_t, tables_t)


def kernel(inputs, tables):
    return _run(inputs.T, tables.transpose(0, 2, 1))


# speedup vs baseline: 5.0071x; 5.0071x over previous
import jax
import jax.numpy as jnp
from jax import lax
from jax.experimental import pallas as pl
from jax.experimental.pallas import tpu as pltpu
from jax.experimental.pallas import tpu_sc as plsc

NUM_SPARSE = 26
HIST_LEN = 50
VOCAB = 1000000
BATCH = 16384
NUM_FIELDS = NUM_SPARSE + HIST_LEN
NUM_TABLES = NUM_SPARSE + 1

L = 16
NW = 32
B_PER_W = BATCH // NW


def _logit_kernel(*refs):
    inputs_t_hbm = refs[0]
    table_refs = refs[1:1 + NUM_TABLES]
    out_hbm = refs[1 + NUM_TABLES]
    idx_t, vals, outbuf, sem = refs[2 + NUM_TABLES:]

    wid = lax.axis_index("s") * 2 + lax.axis_index("c")
    base = wid * B_PER_W

    pltpu.sync_copy(inputs_t_hbm.at[:, pl.ds(base, B_PER_W)], idx_t)

    copies = []
    for c in range(NUM_FIELDS):
        t = min(c, NUM_SPARSE)
        copies.append(pltpu.async_copy(
            table_refs[t].at[idx_t.at[c]],
            vals.at[c],
            sem))
    for cp in copies:
        cp.wait()

    def rbody(v, carry):
        o = v * L
        acc = jnp.zeros((L,), jnp.float32)
        for c in range(NUM_SPARSE):
            acc = acc + vals[c, pl.ds(o, L)]
        for c in range(NUM_SPARSE, NUM_FIELDS):
            val = vals[c, pl.ds(o, L)]
            raw = idx_t[c, pl.ds(o, L)]
            acc = acc + jnp.where(raw != 0, val, 0.0)
        outbuf[pl.ds(o, L)] = acc
        return carry

    lax.fori_loop(0, B_PER_W // L, rbody, 0)

    pltpu.sync_copy(outbuf, out_hbm.at[pl.ds(base, B_PER_W)])


@jax.jit
def _run(inputs_t, *tables_1d):
    mesh = plsc.VectorSubcoreMesh(core_axis_name="c", subcore_axis_name="s")
    return pl.kernel(
        _logit_kernel,
        mesh=mesh,
        compiler_params=pltpu.CompilerParams(
            needs_layout_passes=False, use_tc_tiling_on_sc=False),
        out_type=jax.ShapeDtypeStruct((BATCH,), jnp.float32),
        scratch_types=[
            pltpu.VMEM((NUM_FIELDS, B_PER_W), jnp.int32),
            pltpu.VMEM((NUM_FIELDS, B_PER_W), jnp.float32),
            pltpu.VMEM((B_PER_W,), jnp.float32),
            pltpu.SemaphoreType.DMA,
        ],
    )(inputs_t, *tables_1d)


def kernel(inputs, tables):
    tables_1d = tuple(tables[t, :, 0] for t in range(NUM_TABLES))
    return _run(inputs.T, *tables_1d)
